# baseline (speedup 1.0000x reference)
import functools

import jax
import jax.numpy as jnp
from jax import lax
from jax.experimental import pallas as pl
from jax.experimental.pallas import tpu as pltpu
from jax.experimental.pallas import tpu_sc as plsc

NC = 2
NS = 16
NW = NC * NS
BATCH = 16384
LATENT = 64
CHUNK = BATCH // NW
NIDX = 4
IDXW = CHUNK // NIDX
NBLK = CHUNK // 16

_mesh = plsc.VectorSubcoreMesh(core_axis_name="c", subcore_axis_name="s")


_scratch_types = [
    pltpu.VMEM((NIDX, IDXW), jnp.int32),
    pltpu.VMEM((NIDX, IDXW), jnp.int32),
    pltpu.VMEM((NIDX, IDXW, LATENT), jnp.float32),
    pltpu.VMEM((NIDX, IDXW, LATENT), jnp.float32),
    pltpu.VMEM((NIDX, IDXW), jnp.float32),
    pltpu.VMEM((NIDX, IDXW), jnp.float32),
    pltpu.VMEM((CHUNK,), jnp.float32),
    pltpu.SemaphoreType.DMA,
]


def _mf_body(users_hbm, items_hbm, p_hbm, q_hbm, bu_hbm, bi_hbm, out_hbm,
             uidx, iidx, pm, qm, bu, bi, outb, sem):
    wid = lax.axis_index("s") * NC + lax.axis_index("c")

    pltpu.sync_copy(users_hbm.at[wid], uidx)
    pltpu.sync_copy(items_hbm.at[wid], iidx)

    copies = []
    for j in range(NIDX):
        copies.append(pltpu.async_copy(p_hbm.at[uidx.at[j]], pm.at[j], sem))
        copies.append(pltpu.async_copy(q_hbm.at[iidx.at[j]], qm.at[j], sem))
        copies.append(pltpu.async_copy(bu_hbm.at[uidx.at[j]], bu.at[j], sem))
        copies.append(pltpu.async_copy(bi_hbm.at[iidx.at[j]], bi.at[j], sem))
    for c in copies:
        c.wait()

    lane = lax.iota(jnp.int32, 16)
    perms = [lane ^ s for s in (1, 2, 4, 8)]
    picks = [(lane & s) == 0 for s in (1, 2, 4, 8)]

    def _take(v, idx):
        return jnp.take_along_axis(v, idx, axis=0)

    def blk_body(blk, carry):
        j = blk // (IDXW // 16)
        off = (blk % (IDXW // 16)) * 16
        vs = []
        for r in range(16):
            row = off + r
            acc = pm[j, row, pl.ds(0, 16)] * qm[j, row, pl.ds(0, 16)]
            for k in range(1, LATENT // 16):
                acc = acc + pm[j, row, pl.ds(16 * k, 16)] * qm[j, row, pl.ds(16 * k, 16)]
            vs.append(acc)
        for lvl in range(4):
            nxt = []
            for i in range(0, len(vs), 2):
                a, b = vs[i], vs[i + 1]
                ap = a + _take(a, perms[lvl])
                bp = b + _take(b, perms[lvl])
                nxt.append(jnp.where(picks[lvl], ap, bp))
            vs = nxt
        out_v = vs[0] + bu[j, pl.ds(off, 16)] + bi[j, pl.ds(off, 16)]
        outb[pl.ds(blk * 16, 16)] = out_v
        return carry

    lax.fori_loop(0, NBLK, blk_body, 0)

    pltpu.sync_copy(outb, out_hbm.at[pl.ds(wid * CHUNK, CHUNK)])


_mf_kernel = functools.partial(
    pl.kernel,
    out_type=jax.ShapeDtypeStruct((BATCH,), jnp.float32),
    mesh=_mesh,
    scratch_types=_scratch_types,
    compiler_params=pltpu.CompilerParams(use_tc_tiling_on_sc=False),
)(_mf_body)


def kernel(users, items, P, Q, user_bias, item_bias):
    users_r = users.reshape(NW, NIDX, IDXW)
    items_r = items.reshape(NW, NIDX, IDXW)
    bu_flat = user_bias.reshape(-1)
    bi_flat = item_bias.reshape(-1)
    return _mf_kernel(users_r, items_r, P, Q, bu_flat, bi_flat)
